# baseline (speedup 1.0000x reference)
import functools

import jax
import jax.numpy as jnp
from jax import lax
from jax.experimental import pallas as pl
from jax.experimental.pallas import tpu as pltpu
from jax.experimental.pallas import tpu_sc as plsc

BATCH = 16384
INPUT_DIM = 39
N_CONT = 13
N_CAT = 26
VOCAB = 100000
EMB = 64
OUT_DIM = N_CONT + N_CAT * EMB

NUM_CORES = 2
NUM_SUBCORES = 16
NW = NUM_CORES * NUM_SUBCORES
BPW = BATCH // NW
CB = 32
NPAIR = BPW // (2 * CB)
NB = 2
TABW = 2 * EMB
LANES = 16
GRP = CB // LANES


def _body(xt_hbm, tab_hbm, out_hbm, xv, idxv, st0, st1, rows, sems, wsems):
    wid = lax.axis_index("s") * NUM_CORES + lax.axis_index("c")
    base = wid * BPW
    lanes = lax.iota(jnp.int32, LANES)
    stages = [st0, st1]

    def place(c, buf, stage):
        col0 = N_CONT + c * EMB

        def place_body(r, icarry):
            rvec = jnp.full((LANES,), r, jnp.int32)
            for k in range(EMB // LANES):
                v = plsc.load_gather(buf, [rvec, k * LANES + lanes])
                plsc.store_scatter(stage, [rvec, col0 + k * LANES + lanes], v)
            return icarry

        lax.fori_loop(0, CB, place_body, 0)

    def do_chunk(p, half, reclaim):
        stage = stages[half]
        row0 = base + (2 * p + half) * CB

        pltpu.sync_copy(xt_hbm.at[:, pl.ds(row0, CB)], xv)

        for c in range(N_CAT):
            def idx_body(g, icarry, c=c):
                v = xv[N_CONT + c, pl.ds(g * LANES, LANES)] + c * VOCAB
                row = jnp.full((LANES,), c, jnp.int32)
                plsc.store_scatter(idxv, [row, g * LANES + lanes], v)
                return icarry

            lax.fori_loop(0, GRP, idx_body, 0)

        def gather(c):
            return pltpu.async_copy(
                tab_hbm.at[idxv.at[c]], rows[c % NB], sems[c % NB])

        copies = [gather(c) for c in range(NB)]

        if reclaim:
            pltpu.make_async_copy(
                stage, out_hbm.at[pl.ds(row0, CB)], wsems[half]).wait()

        for j in range(N_CONT):
            def cont_body(g, icarry, j=j):
                v = xv[j, pl.ds(g * LANES, LANES)]
                plsc.store_scatter(
                    stage,
                    [g * LANES + lanes, jnp.full((LANES,), j, jnp.int32)],
                    v.astype(jnp.float32))
                return icarry

            lax.fori_loop(0, GRP, cont_body, 0)

        for c in range(N_CAT):
            copies[c % NB].wait()
            place(c, rows[c % NB], stage)
            if c + NB < N_CAT:
                copies[(c + NB) % NB] = gather(c + NB)

        pltpu.make_async_copy(
            stage, out_hbm.at[pl.ds(row0, CB)], wsems[half]).start()

    do_chunk(0, 0, False)
    do_chunk(0, 1, False)

    def pair_body(p, carry):
        do_chunk(p, 0, True)
        do_chunk(p, 1, True)
        return carry

    lax.fori_loop(1, NPAIR, pair_body, 0)

    for half in range(2):
        row0 = base + (2 * (NPAIR - 1) + half) * CB
        pltpu.make_async_copy(
            stages[half], out_hbm.at[pl.ds(row0, CB)], wsems[half]).wait()


@jax.jit
def _run(xt, tab_flat):
    mesh = plsc.VectorSubcoreMesh(core_axis_name="c", subcore_axis_name="s")
    kern = functools.partial(
        pl.kernel,
        out_type=jax.ShapeDtypeStruct((BATCH, OUT_DIM), jnp.float32),
        mesh=mesh,
        compiler_params=pltpu.CompilerParams(
            use_tc_tiling_on_sc=False, needs_layout_passes=False),
        scratch_types=[
            pltpu.VMEM((INPUT_DIM, CB), jnp.int32),
            pltpu.VMEM((N_CAT, CB), jnp.int32),
            pltpu.VMEM((CB, OUT_DIM), jnp.float32),
            pltpu.VMEM((CB, OUT_DIM), jnp.float32),
            [pltpu.VMEM((CB, TABW), jnp.float32)] * NB,
            [pltpu.SemaphoreType.DMA] * NB,
            [pltpu.SemaphoreType.DMA] * 2,
        ],
    )(_body)
    return kern(xt, tab_flat)


def kernel(x, tables):
    xt = x.T
    tab128 = jnp.pad(tables.reshape(N_CAT * VOCAB, EMB), ((0, 0), (0, EMB)))
    return _run(xt, tab128)


# speedup vs baseline: 1.1257x; 1.1257x over previous
import functools

import jax
import jax.numpy as jnp
from jax import lax
from jax.experimental import pallas as pl
from jax.experimental.pallas import tpu as pltpu
from jax.experimental.pallas import tpu_sc as plsc

BATCH = 16384
INPUT_DIM = 39
N_CONT = 13
N_CAT = 26
VOCAB = 100000
EMB = 64
OUT_DIM = N_CONT + N_CAT * EMB

NUM_CORES = 2
NUM_SUBCORES = 16
NW = NUM_CORES * NUM_SUBCORES
BPW = BATCH // NW
CB = 64
NCH = BPW // CB
NB = 2
TABW = 2 * EMB
LANES = 16
GRP = CB // LANES


def _body(xt_hbm, tab_hbm, out_hbm, xv, idxv, stage, rows, sems):
    wid = lax.axis_index("s") * NUM_CORES + lax.axis_index("c")
    base = wid * BPW
    lanes = lax.iota(jnp.int32, LANES)

    def place(c, buf):
        col0 = N_CONT + c * EMB

        def place_body(r, icarry):
            rvec = jnp.full((LANES,), r, jnp.int32)
            for k in range(EMB // LANES):
                v = plsc.load_gather(buf, [rvec, k * LANES + lanes])
                plsc.store_scatter(stage, [rvec, col0 + k * LANES + lanes], v)
            return icarry

        lax.fori_loop(0, CB, place_body, 0)

    def chunk_body(ch, carry):
        row0 = base + ch * CB

        pltpu.sync_copy(xt_hbm.at[:, pl.ds(row0, CB)], xv)

        for c in range(N_CAT):
            def idx_body(g, icarry, c=c):
                v = xv[N_CONT + c, pl.ds(g * LANES, LANES)] + c * VOCAB
                row = jnp.full((LANES,), c, jnp.int32)
                plsc.store_scatter(idxv, [row, g * LANES + lanes], v)
                return icarry

            lax.fori_loop(0, GRP, idx_body, 0)

        for j in range(N_CONT):
            def cont_body(g, icarry, j=j):
                v = xv[j, pl.ds(g * LANES, LANES)]
                plsc.store_scatter(
                    stage,
                    [g * LANES + lanes, jnp.full((LANES,), j, jnp.int32)],
                    v.astype(jnp.float32))
                return icarry

            lax.fori_loop(0, GRP, cont_body, 0)

        def gather(c):
            return pltpu.async_copy(
                tab_hbm.at[idxv.at[c]], rows[c % NB], sems[c % NB])

        copies = [gather(c) for c in range(NB)]
        for c in range(N_CAT):
            copies[c % NB].wait()
            place(c, rows[c % NB])
            if c + NB < N_CAT:
                copies[(c + NB) % NB] = gather(c + NB)

        pltpu.sync_copy(stage, out_hbm.at[pl.ds(row0, CB)])
        return carry

    lax.fori_loop(0, NCH, chunk_body, 0)


@jax.jit
def _run(xt, tab_flat):
    mesh = plsc.VectorSubcoreMesh(core_axis_name="c", subcore_axis_name="s")
    kern = functools.partial(
        pl.kernel,
        out_type=jax.ShapeDtypeStruct((BATCH, OUT_DIM), jnp.float32),
        mesh=mesh,
        compiler_params=pltpu.CompilerParams(
            use_tc_tiling_on_sc=False, needs_layout_passes=False),
        scratch_types=[
            pltpu.VMEM((INPUT_DIM, CB), jnp.int32),
            pltpu.VMEM((N_CAT, CB), jnp.int32),
            pltpu.VMEM((CB, OUT_DIM), jnp.float32),
            [pltpu.VMEM((CB, TABW), jnp.float32)] * NB,
            [pltpu.SemaphoreType.DMA] * NB,
        ],
    )(_body)
    return kern(xt, tab_flat)


def kernel(x, tables):
    xt = x.T
    tab128 = jnp.pad(tables.reshape(N_CAT * VOCAB, EMB), ((0, 0), (0, EMB)))
    return _run(xt, tab128)
